# baseline (speedup 1.0000x reference)
import jax
import jax.numpy as jnp
from jax import lax
from jax.experimental import pallas as pl
from jax.experimental.pallas import tpu as pltpu
from jax.experimental.pallas import tpu_sc as plsc

N = 10000
E = 320000
DF = 128
DE = 16
PD = 3
H = 128
G = 64

NP = 10240
DZ = 144
FW = 24
NC = 2
NS = 16
NW = NC * NS
EWP = 10240
EP = NW * EWP
CH1 = 64
CPW1 = EWP // CH1
CH = 128
CPW = EWP // CH
EPP = EP + 2 * CH
RPT = NP // NS
BN = 1024
GRID = NP // BN


def _make_sc_body(width, with_ea):
    ch = CH1 if with_ea else CH
    cpw = EWP // ch

    def body(refs):
        if with_ea:
            (z_hbm, ea_hbm, sd_hbm, zz_hbm, zze_hbm,
             uz_out, ue_out,
             sd0, sd1, rows0, rows1, ea0, ea1, acc, acce,
             sg0, sg1, si0, si1) = refs
            eab = (ea0, ea1)
        else:
            (z_hbm, sd_hbm, zz_hbm,
             uz_out,
             sd0, sd1, rows0, rows1, acc,
             sg0, sg1, si0, si1) = refs
        cid = lax.axis_index("c")
        sid = lax.axis_index("s")
        wid = sid * NC + cid
        base = sid * RPT
        pltpu.sync_copy(zz_hbm.at[pl.ds(base, RPT)], acc.at[pl.ds(base, RPT)])
        if with_ea:
            pltpu.sync_copy(zze_hbm.at[pl.ds(base, RPT)],
                            acce.at[pl.ds(base, RPT)])
        ebase = wid * EWP
        cbase = wid * cpw
        sd = (sd0, sd1)
        rows = (rows0, rows1)
        sg = (sg0, sg1)
        si = (si0, si1)

        def eoff(j):
            return pl.multiple_of(ebase + j * ch, 8)

        def idx_start(j, b):
            pltpu.async_copy(sd_hbm.at[cbase + j], sd[b], si[b])

        def idx_wait(j, b):
            pltpu.make_async_copy(sd_hbm.at[cbase + j], sd[b], si[b]).wait()

        def gather_start(j, b):
            pltpu.async_copy(z_hbm.at[sd[b].at[0]], rows[b], sg[b])
            if with_ea:
                pltpu.async_copy(ea_hbm.at[pl.ds(eoff(j), ch)], eab[b], sg[b])

        def gather_wait(j, b):
            pltpu.make_async_copy(z_hbm.at[sd[b].at[0]], rows[b], sg[b]).wait()
            if with_ea:
                pltpu.make_async_copy(ea_hbm.at[pl.ds(eoff(j), ch)],
                                      eab[b], sg[b]).wait()

        def scatter(j, b):
            pltpu.sync_copy(rows[b], acc.at[sd[b].at[1]], add=True)
            if with_ea:
                pltpu.sync_copy(eab[b], acce.at[sd[b].at[1]], add=True)

        pltpu.sync_copy(sd_hbm.at[cbase], sd[0])
        gather_start(0, 0)
        idx_start(1, 1)

        def step(j, b):
            gather_wait(j, b)
            idx_wait(j + 1, 1 - b)
            gather_start(j + 1, 1 - b)
            scatter(j, b)
            idx_start(j + 2, b)

        def pair(i, carry):
            step(2 * i, 0)
            step(2 * i + 1, 1)
            return carry

        lax.fori_loop(0, cpw // 2, pair, 0)
        gather_wait(cpw, 0)
        idx_wait(cpw + 1, 1)
        plsc.subcore_barrier()
        pltpu.sync_copy(acc.at[pl.ds(base, RPT)],
                        uz_out.at[cid, pl.ds(base, RPT)])
        if with_ea:
            pltpu.sync_copy(acce.at[pl.ds(base, RPT)],
                            ue_out.at[cid, pl.ds(base, RPT)])

    def wrapped(*refs):
        return body(refs)

    return wrapped


import functools


@functools.cache
def _sc_kernels():
    mesh = plsc.VectorSubcoreMesh(core_axis_name="c", subcore_axis_name="s")
    params = pltpu.CompilerParams(use_tc_tiling_on_sc=False)
    sc_agg1 = pl.kernel(
        _make_sc_body(DZ, True),
        mesh=mesh,
        compiler_params=params,
        out_type=[jax.ShapeDtypeStruct((NC, NP, DZ), jnp.float32),
                  jax.ShapeDtypeStruct((NC, NP, DE), jnp.float32)],
        scratch_types=[
            pltpu.VMEM((2, CH1), jnp.int32),
            pltpu.VMEM((2, CH1), jnp.int32),
            pltpu.VMEM((CH1, DZ), jnp.float32),
            pltpu.VMEM((CH1, DZ), jnp.float32),
            pltpu.VMEM((CH1, DE), jnp.float32),
            pltpu.VMEM((CH1, DE), jnp.float32),
            pltpu.VMEM_SHARED((NP, DZ), jnp.float32),
            pltpu.VMEM_SHARED((NP, DE), jnp.float32),
            pltpu.SemaphoreType.DMA,
            pltpu.SemaphoreType.DMA,
            pltpu.SemaphoreType.DMA,
            pltpu.SemaphoreType.DMA,
        ],
    )
    sc_aggh = pl.kernel(
        _make_sc_body(H, False),
        mesh=mesh,
        compiler_params=params,
        out_type=jax.ShapeDtypeStruct((NC, NP, H), jnp.float32),
        scratch_types=[
            pltpu.VMEM((2, CH), jnp.int32),
            pltpu.VMEM((2, CH), jnp.int32),
            pltpu.VMEM((CH, H), jnp.float32),
            pltpu.VMEM((CH, H), jnp.float32),
            pltpu.VMEM_SHARED((NP, H), jnp.float32),
            pltpu.SemaphoreType.DMA,
            pltpu.SemaphoreType.DMA,
            pltpu.SemaphoreType.DMA,
            pltpu.SemaphoreType.DMA,
        ],
    )
    return sc_agg1, sc_aggh


def _dot(a, b):
    return jnp.dot(a, b, preferred_element_type=jnp.float32)


def _tc1_body(z_ref, u0_ref, u1_ref, e0_ref, e1_ref,
              wx_ref, v_ref, wr_ref, br_ref,
              h1_ref, f_ref):
    u = u0_ref[...] + u1_ref[...]
    se = e0_ref[...] + e1_ref[...]
    xb = z_ref[:, :DF]
    posb = z_ref[:, DF:DF + PD]
    deg = u[:, DF + PD:DF + PD + 1]
    invd = 1.0 / jnp.maximum(deg, 1.0)
    m = jnp.where(deg > 0, 1.0, 0.0)
    sx = u[:, :DF]
    sp = u[:, DF:DF + PD]
    zero3 = jnp.zeros_like(posb)
    f = jnp.concatenate(
        [se * invd, (sp - deg * posb) * invd, m, invd, zero3], axis=1)
    agg = _dot(sx, wx_ref[...]) * invd + _dot(f, v_ref[...])
    h1_ref[...] = jnp.maximum(_dot(xb, wr_ref[...]) + br_ref[...] + agg, 0.0)
    f_ref[...] = f


def _tc2_body(h_ref, u0_ref, u1_ref, f_ref,
              wx_ref, v_ref, wr_ref, br_ref,
              ho_ref):
    u = u0_ref[...] + u1_ref[...]
    f = f_ref[...]
    agg = _dot(u, wx_ref[...]) * f[:, FW - 4:FW - 3] + _dot(f, v_ref[...])
    ho_ref[...] = jnp.maximum(
        _dot(h_ref[...], wr_ref[...]) + br_ref[...] + agg, 0.0)


def _tc3_body(h_ref, u0_ref, u1_ref, f_ref, batch_ref,
              wx_ref, v_ref, wr_ref, br_ref,
              w1_ref, b1_ref, w2_ref, b2_ref, w3_ref, b3_ref,
              out_ref, p_acc, c_acc):
    i = pl.program_id(0)
    u = u0_ref[...] + u1_ref[...]
    f = f_ref[...]
    agg = _dot(u, wx_ref[...]) * f[:, FW - 4:FW - 3] + _dot(f, v_ref[...])
    h3 = jnp.maximum(_dot(h_ref[...], wr_ref[...]) + br_ref[...] + agg, 0.0)

    @pl.when(i == 0)
    def _():
        p_acc[...] = jnp.zeros_like(p_acc)
        c_acc[...] = jnp.zeros_like(c_acc)

    rowid = i * BN + lax.broadcasted_iota(jnp.int32, (BN, 1), 0)
    valid = rowid < N
    bb = batch_ref[...]
    gid = lax.broadcasted_iota(jnp.int32, (1, G), 1).astype(jnp.float32)
    onehot = jnp.where((bb == gid) & valid, 1.0, 0.0)
    dimnum = (((0,), (0,)), ((), ()))
    p_acc[...] += lax.dot_general(onehot, h3, dimnum,
                                  preferred_element_type=jnp.float32)
    c_acc[...] += lax.dot_general(onehot, jnp.ones_like(h3), dimnum,
                                  preferred_element_type=jnp.float32)

    @pl.when(i == GRID - 1)
    def _():
        g = p_acc[...] / jnp.maximum(c_acc[...], 1.0)
        g = jnp.maximum(_dot(g, w1_ref[...]) + b1_ref[...], 0.0)
        g = jnp.maximum(_dot(g, w2_ref[...]) + b2_ref[...], 0.0)
        lg = _dot(g, w3_ref[...]) + b3_ref[...]
        s = lg - jnp.max(lg, axis=1, keepdims=True)
        out_ref[...] = s - jnp.log(jnp.sum(jnp.exp(s), axis=1, keepdims=True))


def _row_spec(width):
    return pl.BlockSpec((BN, width), lambda i: (i, 0))


def _whole(shape):
    ndim = len(shape)
    return pl.BlockSpec(shape, lambda i: (0,) * ndim)


def _tc1(z, u0, u1, e0, e1, wx, v, wr, br):
    return pl.pallas_call(
        _tc1_body,
        grid=(GRID,),
        in_specs=[_row_spec(DZ), _row_spec(DZ), _row_spec(DZ),
                  _row_spec(DE), _row_spec(DE),
                  _whole((DF, H)), _whole((FW, H)), _whole((DF, H)),
                  _whole((1, H))],
        out_specs=[_row_spec(H), _row_spec(FW)],
        out_shape=[jax.ShapeDtypeStruct((NP, H), jnp.float32),
                   jax.ShapeDtypeStruct((NP, FW), jnp.float32)],
    )(z, u0, u1, e0, e1, wx, v, wr, br)


def _tc2(h, u0, u1, f, wx, v, wr, br):
    return pl.pallas_call(
        _tc2_body,
        grid=(GRID,),
        in_specs=[_row_spec(H), _row_spec(H), _row_spec(H), _row_spec(FW),
                  _whole((H, H)), _whole((FW, H)), _whole((H, H)),
                  _whole((1, H))],
        out_specs=_row_spec(H),
        out_shape=jax.ShapeDtypeStruct((NP, H), jnp.float32),
    )(h, u0, u1, f, wx, v, wr, br)


def _tc3(h, u0, u1, f, batchf, wx, v, wr, br, w1, b1, w2, b2, w3, b3):
    return pl.pallas_call(
        _tc3_body,
        grid=(GRID,),
        in_specs=[_row_spec(H), _row_spec(H), _row_spec(H), _row_spec(FW),
                  _row_spec(1),
                  _whole((H, H)), _whole((FW, H)), _whole((H, H)),
                  _whole((1, H)),
                  _whole((H, H)), _whole((1, H)), _whole((H, H)),
                  _whole((1, H)), _whole((H, 2)), _whole((1, 2))],
        out_specs=_whole((G, 2)),
        out_shape=jax.ShapeDtypeStruct((G, 2), jnp.float32),
        scratch_shapes=[pltpu.VMEM((G, H), jnp.float32),
                        pltpu.VMEM((G, H), jnp.float32)],
    )(h, u0, u1, f, batchf, wx, v, wr, br, w1, b1, w2, b2, w3, b3)


def _mk_v(wm, bm):
    return jnp.concatenate(
        [wm[DF:], bm[None, :], jnp.zeros((FW - DE - PD - 1, H), jnp.float32)],
        axis=0)


def kernel(x, edge_index, batch, edge_attr, pos,
           Wm1, bm1, Wr1, br1, Wm2, bm2, Wr2, br2, Wm3, bm3, Wr3, br3,
           W1, b1, W2, b2, W3, b3):
    f32 = jnp.float32
    padidx = (N + jnp.arange(EPP - E, dtype=jnp.int32) % (NP - N))
    src = jnp.concatenate([edge_index[0].astype(jnp.int32), padidx])
    dst = jnp.concatenate([edge_index[1].astype(jnp.int32), padidx])

    def _sd(ch):
        n = EP + 2 * ch
        return jnp.concatenate([src[:n].reshape(-1, 1, ch),
                                dst[:n].reshape(-1, 1, ch)], axis=1)

    sd1, sdh = _sd(CH1), _sd(CH)
    ea = jnp.pad(edge_attr, ((0, EP + 2 * CH1 - E), (0, 0)))

    z = jnp.concatenate(
        [x, pos, jnp.ones((N, 1), f32), jnp.zeros((N, DZ - DF - PD - 1), f32)],
        axis=1)
    z = jnp.pad(z, ((0, NP - N), (0, 0)))
    zz = jnp.zeros((NP, DZ), f32)
    zze = jnp.zeros((NP, DE), f32)
    zzh = jnp.zeros((NP, H), f32)
    batchf = jnp.pad(batch.astype(f32), (0, NP - N)).reshape(NP, 1)

    wx1, wx2, wx3 = Wm1[:DF], Wm2[:H], Wm3[:H]
    v1, v2, v3 = _mk_v(Wm1, bm1), _mk_v(Wm2, bm2), _mk_v(Wm3, bm3)
    br1r, br2r, br3r = br1[None, :], br2[None, :], br3[None, :]

    sc_agg1, sc_aggh = _sc_kernels()
    uz, ue = sc_agg1(z, ea, sd1, zz, zze)
    h1, f = _tc1(z, uz[0], uz[1], ue[0], ue[1], wx1, v1, Wr1, br1r)
    u2 = sc_aggh(h1, sdh, zzh)
    h2 = _tc2(h1, u2[0], u2[1], f, wx2, v2, Wr2, br2r)
    u3 = sc_aggh(h2, sdh, zzh)
    return _tc3(h2, u3[0], u3[1], f, batchf, wx3, v3, Wr3, br3r,
                W1, b1[None, :], W2, b2[None, :], W3, b3[None, :])


# speedup vs baseline: 1.0217x; 1.0217x over previous
import jax
import jax.numpy as jnp
from jax import lax
from jax.experimental import pallas as pl
from jax.experimental.pallas import tpu as pltpu
from jax.experimental.pallas import tpu_sc as plsc

N = 10000
E = 320000
DF = 128
DE = 16
PD = 3
H = 128
G = 64

NP = 10240
DZ = 144
FW = 24
NC = 2
NS = 16
NW = NC * NS
EWP = 10240
EP = NW * EWP
CH1 = 64
CPW1 = EWP // CH1
CH = 128
CPW = EWP // CH
EPP = EP + 2 * CH
RPT = NP // NS
BN = 1024
GRID = NP // BN


def _make_sc_body(width, with_ea):
    ch = CH1 if with_ea else CH
    cpw = EWP // ch

    def body(refs):
        if with_ea:
            (z_hbm, ea_hbm, src_hbm, dst_hbm, zz_hbm, zze_hbm,
             uz_out, ue_out,
             is0, is1, id0, id1, rows0, rows1, ea0, ea1, acc, acce,
             sg0, sg1, si0, si1) = refs
            eab = (ea0, ea1)
        else:
            (z_hbm, src_hbm, dst_hbm, zz_hbm,
             uz_out,
             is0, is1, id0, id1, rows0, rows1, acc,
             sg0, sg1, si0, si1) = refs
        cid = lax.axis_index("c")
        sid = lax.axis_index("s")
        wid = sid * NC + cid
        base = sid * RPT
        pltpu.sync_copy(zz_hbm.at[pl.ds(base, RPT)], acc.at[pl.ds(base, RPT)])
        if with_ea:
            pltpu.sync_copy(zze_hbm.at[pl.ds(base, RPT)],
                            acce.at[pl.ds(base, RPT)])
        ebase = wid * EWP
        idxs = (is0, is1)
        idxd = (id0, id1)
        rows = (rows0, rows1)
        sg = (sg0, sg1)
        si = (si0, si1)

        def eoff(j):
            return pl.multiple_of(ebase + j * ch, 8)

        def idx_start(j, b):
            pltpu.async_copy(src_hbm.at[pl.ds(eoff(j), ch)], idxs[b], si[b])
            pltpu.async_copy(dst_hbm.at[pl.ds(eoff(j), ch)], idxd[b], si[b])

        def idx_wait(j, b):
            pltpu.make_async_copy(src_hbm.at[pl.ds(eoff(j), ch)],
                                  idxs[b], si[b]).wait()
            pltpu.make_async_copy(dst_hbm.at[pl.ds(eoff(j), ch)],
                                  idxd[b], si[b]).wait()

        def gather_start(j, b):
            pltpu.async_copy(z_hbm.at[idxs[b]], rows[b], sg[b])
            if with_ea:
                pltpu.async_copy(ea_hbm.at[pl.ds(eoff(j), ch)], eab[b], sg[b])

        def gather_wait(j, b):
            pltpu.make_async_copy(z_hbm.at[idxs[b]], rows[b], sg[b]).wait()
            if with_ea:
                pltpu.make_async_copy(ea_hbm.at[pl.ds(eoff(j), ch)],
                                      eab[b], sg[b]).wait()

        def scatter(j, b):
            pltpu.sync_copy(rows[b], acc.at[idxd[b]], add=True)
            if with_ea:
                pltpu.sync_copy(eab[b], acce.at[idxd[b]], add=True)

        pltpu.sync_copy(src_hbm.at[pl.ds(eoff(0), ch)], idxs[0])
        pltpu.sync_copy(dst_hbm.at[pl.ds(eoff(0), ch)], idxd[0])
        gather_start(0, 0)
        idx_start(1, 1)

        def step(j, b):
            gather_wait(j, b)
            idx_wait(j + 1, 1 - b)
            gather_start(j + 1, 1 - b)
            scatter(j, b)
            idx_start(j + 2, b)

        def pair(i, carry):
            step(2 * i, 0)
            step(2 * i + 1, 1)
            return carry

        lax.fori_loop(0, cpw // 2, pair, 0)
        gather_wait(cpw, 0)
        idx_wait(cpw + 1, 1)
        plsc.subcore_barrier()
        pltpu.sync_copy(acc.at[pl.ds(base, RPT)],
                        uz_out.at[cid, pl.ds(base, RPT)])
        if with_ea:
            pltpu.sync_copy(acce.at[pl.ds(base, RPT)],
                            ue_out.at[cid, pl.ds(base, RPT)])

    def wrapped(*refs):
        return body(refs)

    return wrapped


import functools


@functools.cache
def _sc_kernels():
    mesh = plsc.VectorSubcoreMesh(core_axis_name="c", subcore_axis_name="s")
    params = pltpu.CompilerParams(use_tc_tiling_on_sc=False)
    sc_agg1 = pl.kernel(
        _make_sc_body(DZ, True),
        mesh=mesh,
        compiler_params=params,
        out_type=[jax.ShapeDtypeStruct((NC, NP, DZ), jnp.float32),
                  jax.ShapeDtypeStruct((NC, NP, DE), jnp.float32)],
        scratch_types=[
            pltpu.VMEM((CH1,), jnp.int32),
            pltpu.VMEM((CH1,), jnp.int32),
            pltpu.VMEM((CH1,), jnp.int32),
            pltpu.VMEM((CH1,), jnp.int32),
            pltpu.VMEM((CH1, DZ), jnp.float32),
            pltpu.VMEM((CH1, DZ), jnp.float32),
            pltpu.VMEM((CH1, DE), jnp.float32),
            pltpu.VMEM((CH1, DE), jnp.float32),
            pltpu.VMEM_SHARED((NP, DZ), jnp.float32),
            pltpu.VMEM_SHARED((NP, DE), jnp.float32),
            pltpu.SemaphoreType.DMA,
            pltpu.SemaphoreType.DMA,
            pltpu.SemaphoreType.DMA,
            pltpu.SemaphoreType.DMA,
        ],
    )
    sc_aggh = pl.kernel(
        _make_sc_body(H, False),
        mesh=mesh,
        out_type=jax.ShapeDtypeStruct((NC, NP, H), jnp.float32),
        scratch_types=[
            pltpu.VMEM((CH,), jnp.int32),
            pltpu.VMEM((CH,), jnp.int32),
            pltpu.VMEM((CH,), jnp.int32),
            pltpu.VMEM((CH,), jnp.int32),
            pltpu.VMEM((CH, H), jnp.float32),
            pltpu.VMEM((CH, H), jnp.float32),
            pltpu.VMEM_SHARED((NP, H), jnp.float32),
            pltpu.SemaphoreType.DMA,
            pltpu.SemaphoreType.DMA,
            pltpu.SemaphoreType.DMA,
            pltpu.SemaphoreType.DMA,
        ],
    )
    return sc_agg1, sc_aggh


def _dot(a, b):
    return jnp.dot(a, b, preferred_element_type=jnp.float32)


def _tc1_body(z_ref, u0_ref, u1_ref, e0_ref, e1_ref,
              wx_ref, v_ref, wr_ref, br_ref,
              h1_ref, f_ref):
    u = u0_ref[...] + u1_ref[...]
    se = e0_ref[...] + e1_ref[...]
    xb = z_ref[:, :DF]
    posb = z_ref[:, DF:DF + PD]
    deg = u[:, DF + PD:DF + PD + 1]
    invd = 1.0 / jnp.maximum(deg, 1.0)
    m = jnp.where(deg > 0, 1.0, 0.0)
    sx = u[:, :DF]
    sp = u[:, DF:DF + PD]
    zero3 = jnp.zeros_like(posb)
    f = jnp.concatenate(
        [se * invd, (sp - deg * posb) * invd, m, invd, zero3], axis=1)
    agg = _dot(sx, wx_ref[...]) * invd + _dot(f, v_ref[...])
    h1_ref[...] = jnp.maximum(_dot(xb, wr_ref[...]) + br_ref[...] + agg, 0.0)
    f_ref[...] = f


def _tc2_body(h_ref, u0_ref, u1_ref, f_ref,
              wx_ref, v_ref, wr_ref, br_ref,
              ho_ref):
    u = u0_ref[...] + u1_ref[...]
    f = f_ref[...]
    agg = _dot(u, wx_ref[...]) * f[:, FW - 4:FW - 3] + _dot(f, v_ref[...])
    ho_ref[...] = jnp.maximum(
        _dot(h_ref[...], wr_ref[...]) + br_ref[...] + agg, 0.0)


def _tc3_body(h_ref, u0_ref, u1_ref, f_ref, batch_ref,
              wx_ref, v_ref, wr_ref, br_ref,
              w1_ref, b1_ref, w2_ref, b2_ref, w3_ref, b3_ref,
              out_ref, p_acc, c_acc):
    i = pl.program_id(0)
    u = u0_ref[...] + u1_ref[...]
    f = f_ref[...]
    agg = _dot(u, wx_ref[...]) * f[:, FW - 4:FW - 3] + _dot(f, v_ref[...])
    h3 = jnp.maximum(_dot(h_ref[...], wr_ref[...]) + br_ref[...] + agg, 0.0)

    @pl.when(i == 0)
    def _():
        p_acc[...] = jnp.zeros_like(p_acc)
        c_acc[...] = jnp.zeros_like(c_acc)

    rowid = i * BN + lax.broadcasted_iota(jnp.int32, (BN, 1), 0)
    valid = rowid < N
    bb = batch_ref[...]
    gid = lax.broadcasted_iota(jnp.int32, (1, G), 1).astype(jnp.float32)
    onehot = jnp.where((bb == gid) & valid, 1.0, 0.0)
    dimnum = (((0,), (0,)), ((), ()))
    p_acc[...] += lax.dot_general(onehot, h3, dimnum,
                                  preferred_element_type=jnp.float32)
    c_acc[...] += lax.dot_general(onehot, jnp.ones_like(h3), dimnum,
                                  preferred_element_type=jnp.float32)

    @pl.when(i == GRID - 1)
    def _():
        g = p_acc[...] / jnp.maximum(c_acc[...], 1.0)
        g = jnp.maximum(_dot(g, w1_ref[...]) + b1_ref[...], 0.0)
        g = jnp.maximum(_dot(g, w2_ref[...]) + b2_ref[...], 0.0)
        lg = _dot(g, w3_ref[...]) + b3_ref[...]
        s = lg - jnp.max(lg, axis=1, keepdims=True)
        out_ref[...] = s - jnp.log(jnp.sum(jnp.exp(s), axis=1, keepdims=True))


def _row_spec(width):
    return pl.BlockSpec((BN, width), lambda i: (i, 0))


def _whole(shape):
    ndim = len(shape)
    return pl.BlockSpec(shape, lambda i: (0,) * ndim)


def _tc1(z, u0, u1, e0, e1, wx, v, wr, br):
    return pl.pallas_call(
        _tc1_body,
        grid=(GRID,),
        in_specs=[_row_spec(DZ), _row_spec(DZ), _row_spec(DZ),
                  _row_spec(DE), _row_spec(DE),
                  _whole((DF, H)), _whole((FW, H)), _whole((DF, H)),
                  _whole((1, H))],
        out_specs=[_row_spec(H), _row_spec(FW)],
        out_shape=[jax.ShapeDtypeStruct((NP, H), jnp.float32),
                   jax.ShapeDtypeStruct((NP, FW), jnp.float32)],
    )(z, u0, u1, e0, e1, wx, v, wr, br)


def _tc2(h, u0, u1, f, wx, v, wr, br):
    return pl.pallas_call(
        _tc2_body,
        grid=(GRID,),
        in_specs=[_row_spec(H), _row_spec(H), _row_spec(H), _row_spec(FW),
                  _whole((H, H)), _whole((FW, H)), _whole((H, H)),
                  _whole((1, H))],
        out_specs=_row_spec(H),
        out_shape=jax.ShapeDtypeStruct((NP, H), jnp.float32),
    )(h, u0, u1, f, wx, v, wr, br)


def _tc3(h, u0, u1, f, batchf, wx, v, wr, br, w1, b1, w2, b2, w3, b3):
    return pl.pallas_call(
        _tc3_body,
        grid=(GRID,),
        in_specs=[_row_spec(H), _row_spec(H), _row_spec(H), _row_spec(FW),
                  _row_spec(1),
                  _whole((H, H)), _whole((FW, H)), _whole((H, H)),
                  _whole((1, H)),
                  _whole((H, H)), _whole((1, H)), _whole((H, H)),
                  _whole((1, H)), _whole((H, 2)), _whole((1, 2))],
        out_specs=_whole((G, 2)),
        out_shape=jax.ShapeDtypeStruct((G, 2), jnp.float32),
        scratch_shapes=[pltpu.VMEM((G, H), jnp.float32),
                        pltpu.VMEM((G, H), jnp.float32)],
    )(h, u0, u1, f, batchf, wx, v, wr, br, w1, b1, w2, b2, w3, b3)


def _mk_v(wm, bm):
    return jnp.concatenate(
        [wm[DF:], bm[None, :], jnp.zeros((FW - DE - PD - 1, H), jnp.float32)],
        axis=0)


def kernel(x, edge_index, batch, edge_attr, pos,
           Wm1, bm1, Wr1, br1, Wm2, bm2, Wr2, br2, Wm3, bm3, Wr3, br3,
           W1, b1, W2, b2, W3, b3):
    f32 = jnp.float32
    padidx = (N + jnp.arange(EPP - E, dtype=jnp.int32) % (NP - N))
    src = jnp.concatenate([edge_index[0].astype(jnp.int32), padidx])
    dst = jnp.concatenate([edge_index[1].astype(jnp.int32), padidx])

    ea = jnp.pad(edge_attr, ((0, EP + 2 * CH1 - E), (0, 0)))

    z = jnp.concatenate(
        [x, pos, jnp.ones((N, 1), f32), jnp.zeros((N, DZ - DF - PD - 1), f32)],
        axis=1)
    z = jnp.pad(z, ((0, NP - N), (0, 0)))
    zz = jnp.zeros((NP, DZ), f32)
    zze = jnp.zeros((NP, DE), f32)
    zzh = jnp.zeros((NP, H), f32)
    batchf = jnp.pad(batch.astype(f32), (0, NP - N)).reshape(NP, 1)

    wx1, wx2, wx3 = Wm1[:DF], Wm2[:H], Wm3[:H]
    v1, v2, v3 = _mk_v(Wm1, bm1), _mk_v(Wm2, bm2), _mk_v(Wm3, bm3)
    br1r, br2r, br3r = br1[None, :], br2[None, :], br3[None, :]

    sc_agg1, sc_aggh = _sc_kernels()
    uz, ue = sc_agg1(z, ea, src, dst, zz, zze)
    h1, f = _tc1(z, uz[0], uz[1], ue[0], ue[1], wx1, v1, Wr1, br1r)
    u2 = sc_aggh(h1, src, dst, zzh)
    h2 = _tc2(h1, u2[0], u2[1], f, wx2, v2, Wr2, br2r)
    u3 = sc_aggh(h2, src, dst, zzh)
    return _tc3(h2, u3[0], u3[1], f, batchf, wx3, v3, Wr3, br3r,
                W1, b1[None, :], W2, b2[None, :], W3, b3[None, :])


# speedup vs baseline: 1.1865x; 1.1613x over previous
import jax
import jax.numpy as jnp
from jax import lax
from jax.experimental import pallas as pl
from jax.experimental.pallas import tpu as pltpu
from jax.experimental.pallas import tpu_sc as plsc

N = 10000
E = 320000
DF = 128
DE = 16
PD = 3
H = 128
G = 64

NP = 10240
DZ = 144
FW = 24
NC = 2
NS = 16
NW = NC * NS
EWP = 10240
EP = NW * EWP
CH1 = 80
CH = 128
EPP = EP + 2 * CH
RPT = NP // NS
EAT = EWP + 2 * CH
BN = 1024
GRID = NP // BN


def _make_sc_body(ch):
    cpw = EWP // ch

    def body(z_hbm, src_hbm, dst_hbm, zz_hbm, uz_out,
             is0, is1, id0, id1, rows0, rows1, acc,
             sg0, sg1, si0, si1):
        cid = lax.axis_index("c")
        sid = lax.axis_index("s")
        wid = sid * NC + cid
        base = sid * RPT
        pltpu.sync_copy(zz_hbm.at[pl.ds(base, RPT)], acc.at[pl.ds(base, RPT)])
        ebase = wid * EWP
        idxs = (is0, is1)
        idxd = (id0, id1)
        rows = (rows0, rows1)
        sg = (sg0, sg1)
        si = (si0, si1)

        def eoff(j):
            return pl.multiple_of(ebase + j * ch, 8)

        def idx_start(j, b):
            pltpu.async_copy(src_hbm.at[pl.ds(eoff(j), ch)], idxs[b], si[b])
            pltpu.async_copy(dst_hbm.at[pl.ds(eoff(j), ch)], idxd[b], si[b])

        def idx_wait(j, b):
            pltpu.make_async_copy(src_hbm.at[pl.ds(eoff(j), ch)],
                                  idxs[b], si[b]).wait()
            pltpu.make_async_copy(dst_hbm.at[pl.ds(eoff(j), ch)],
                                  idxd[b], si[b]).wait()

        def gather_start(j, b):
            pltpu.async_copy(z_hbm.at[idxs[b]], rows[b], sg[b])

        def gather_wait(j, b):
            pltpu.make_async_copy(z_hbm.at[idxs[b]], rows[b], sg[b]).wait()

        def scatter(j, b):
            pltpu.sync_copy(rows[b], acc.at[idxd[b]], add=True)

        pltpu.sync_copy(src_hbm.at[pl.ds(eoff(0), ch)], idxs[0])
        pltpu.sync_copy(dst_hbm.at[pl.ds(eoff(0), ch)], idxd[0])
        gather_start(0, 0)
        idx_start(1, 1)

        def step(j, b):
            gather_wait(j, b)
            idx_wait(j + 1, 1 - b)
            gather_start(j + 1, 1 - b)
            scatter(j, b)
            idx_start(j + 2, b)

        def pair(i, carry):
            step(2 * i, 0)
            step(2 * i + 1, 1)
            return carry

        lax.fori_loop(0, cpw // 2, pair, 0)
        gather_wait(cpw, 0)
        idx_wait(cpw + 1, 1)
        plsc.subcore_barrier()
        pltpu.sync_copy(acc.at[pl.ds(base, RPT)],
                        uz_out.at[cid, pl.ds(base, RPT)])

    return body


def _sc_ea_body(ea_hbm, eat_hbm, dst_hbm, zze_hbm, ue_out,
                id0, id1, ea0, ea1, acce,
                sg0, sg1, si0, si1):
    cid = lax.axis_index("c")
    sid = lax.axis_index("s")
    wid = sid * NC + cid
    base = sid * RPT
    pltpu.sync_copy(zze_hbm.at[pl.ds(base, RPT)], acce.at[pl.ds(base, RPT)])
    ebase = wid * EWP
    idxd = (id0, id1)
    eab = (ea0, ea1)
    sg = (sg0, sg1)
    si = (si0, si1)
    last = wid == NW - 1

    def eoff(j):
        return pl.multiple_of(ebase + j * CH, 8)

    def toff(j):
        return pl.multiple_of(j * CH, 8)

    def idx_start(j, b):
        pltpu.async_copy(dst_hbm.at[pl.ds(eoff(j), CH)], idxd[b], si[b])

    def idx_wait(j, b):
        pltpu.make_async_copy(dst_hbm.at[pl.ds(eoff(j), CH)],
                              idxd[b], si[b]).wait()

    def ea_start(j, b):
        @pl.when(last)
        def _():
            pltpu.async_copy(eat_hbm.at[pl.ds(toff(j), CH)], eab[b], sg[b])

        @pl.when(jnp.logical_not(last))
        def _():
            pltpu.async_copy(ea_hbm.at[pl.ds(eoff(j), CH)], eab[b], sg[b])

    def ea_wait(j, b):
        @pl.when(last)
        def _():
            pltpu.make_async_copy(eat_hbm.at[pl.ds(toff(j), CH)],
                                  eab[b], sg[b]).wait()

        @pl.when(jnp.logical_not(last))
        def _():
            pltpu.make_async_copy(ea_hbm.at[pl.ds(eoff(j), CH)],
                                  eab[b], sg[b]).wait()

    def scatter(j, b):
        pltpu.sync_copy(eab[b], acce.at[idxd[b]], add=True)

    pltpu.sync_copy(dst_hbm.at[pl.ds(eoff(0), CH)], idxd[0])
    ea_start(0, 0)
    idx_start(1, 1)

    def step(j, b):
        ea_wait(j, b)
        idx_wait(j + 1, 1 - b)
        ea_start(j + 1, 1 - b)
        scatter(j, b)
        idx_start(j + 2, b)

    def pair(i, carry):
        step(2 * i, 0)
        step(2 * i + 1, 1)
        return carry

    lax.fori_loop(0, EWP // CH // 2, pair, 0)
    ea_wait(EWP // CH, 0)
    idx_wait(EWP // CH + 1, 1)
    plsc.subcore_barrier()
    pltpu.sync_copy(acce.at[pl.ds(base, RPT)],
                    ue_out.at[cid, pl.ds(base, RPT)])


import functools


@functools.cache
def _sc_kernels():
    mesh = plsc.VectorSubcoreMesh(core_axis_name="c", subcore_axis_name="s")
    params = pltpu.CompilerParams(use_tc_tiling_on_sc=False)
    sems = [pltpu.SemaphoreType.DMA] * 4
    sc_agg1 = pl.kernel(
        _make_sc_body(CH1),
        mesh=mesh,
        compiler_params=params,
        out_type=jax.ShapeDtypeStruct((NC, NP, DZ), jnp.float32),
        scratch_types=[
            pltpu.VMEM((CH1,), jnp.int32),
            pltpu.VMEM((CH1,), jnp.int32),
            pltpu.VMEM((CH1,), jnp.int32),
            pltpu.VMEM((CH1,), jnp.int32),
            pltpu.VMEM((CH1, DZ), jnp.float32),
            pltpu.VMEM((CH1, DZ), jnp.float32),
            pltpu.VMEM_SHARED((NP, DZ), jnp.float32),
        ] + sems,
    )
    sc_aggh = pl.kernel(
        _make_sc_body(CH),
        mesh=mesh,
        out_type=jax.ShapeDtypeStruct((NC, NP, H), jnp.float32),
        scratch_types=[
            pltpu.VMEM((CH,), jnp.int32),
            pltpu.VMEM((CH,), jnp.int32),
            pltpu.VMEM((CH,), jnp.int32),
            pltpu.VMEM((CH,), jnp.int32),
            pltpu.VMEM((CH, H), jnp.float32),
            pltpu.VMEM((CH, H), jnp.float32),
            pltpu.VMEM_SHARED((NP, H), jnp.float32),
        ] + sems,
    )
    sc_ea = pl.kernel(
        _sc_ea_body,
        mesh=mesh,
        out_type=jax.ShapeDtypeStruct((NC, NP, DE), jnp.float32),
        scratch_types=[
            pltpu.VMEM((CH,), jnp.int32),
            pltpu.VMEM((CH,), jnp.int32),
            pltpu.VMEM((CH, DE), jnp.float32),
            pltpu.VMEM((CH, DE), jnp.float32),
            pltpu.VMEM_SHARED((NP, DE), jnp.float32),
        ] + sems,
    )
    return sc_agg1, sc_aggh, sc_ea


def _dot(a, b):
    return jnp.dot(a, b, preferred_element_type=jnp.float32)


def _tc1_body(z_ref, u0_ref, u1_ref, e0_ref, e1_ref,
              wx_ref, v_ref, wr_ref, br_ref,
              h1_ref, f_ref):
    u = u0_ref[...] + u1_ref[...]
    se = e0_ref[...] + e1_ref[...]
    xb = z_ref[:, :DF]
    posb = z_ref[:, DF:DF + PD]
    deg = u[:, DF + PD:DF + PD + 1]
    invd = 1.0 / jnp.maximum(deg, 1.0)
    m = jnp.where(deg > 0, 1.0, 0.0)
    sx = u[:, :DF]
    sp = u[:, DF:DF + PD]
    zero3 = jnp.zeros_like(posb)
    f = jnp.concatenate(
        [se * invd, (sp - deg * posb) * invd, m, invd, zero3], axis=1)
    agg = _dot(sx, wx_ref[...]) * invd + _dot(f, v_ref[...])
    h1_ref[...] = jnp.maximum(_dot(xb, wr_ref[...]) + br_ref[...] + agg, 0.0)
    f_ref[...] = f


def _tc2_body(h_ref, u0_ref, u1_ref, f_ref,
              wx_ref, v_ref, wr_ref, br_ref,
              ho_ref):
    u = u0_ref[...] + u1_ref[...]
    f = f_ref[...]
    agg = _dot(u, wx_ref[...]) * f[:, FW - 4:FW - 3] + _dot(f, v_ref[...])
    ho_ref[...] = jnp.maximum(
        _dot(h_ref[...], wr_ref[...]) + br_ref[...] + agg, 0.0)


def _tc3_body(h_ref, u0_ref, u1_ref, f_ref, batch_ref,
              wx_ref, v_ref, wr_ref, br_ref,
              w1_ref, b1_ref, w2_ref, b2_ref, w3_ref, b3_ref,
              out_ref, p_acc, c_acc):
    i = pl.program_id(0)
    u = u0_ref[...] + u1_ref[...]
    f = f_ref[...]
    agg = _dot(u, wx_ref[...]) * f[:, FW - 4:FW - 3] + _dot(f, v_ref[...])
    h3 = jnp.maximum(_dot(h_ref[...], wr_ref[...]) + br_ref[...] + agg, 0.0)

    @pl.when(i == 0)
    def _():
        p_acc[...] = jnp.zeros_like(p_acc)
        c_acc[...] = jnp.zeros_like(c_acc)

    rowid = i * BN + lax.broadcasted_iota(jnp.int32, (BN, 1), 0)
    valid = rowid < N
    bb = batch_ref[...]
    gid = lax.broadcasted_iota(jnp.int32, (1, G), 1).astype(jnp.float32)
    onehot = jnp.where((bb == gid) & valid, 1.0, 0.0)
    dimnum = (((0,), (0,)), ((), ()))
    p_acc[...] += lax.dot_general(onehot, h3, dimnum,
                                  preferred_element_type=jnp.float32)
    c_acc[...] += lax.dot_general(onehot, jnp.ones_like(h3), dimnum,
                                  preferred_element_type=jnp.float32)

    @pl.when(i == GRID - 1)
    def _():
        g = p_acc[...] / jnp.maximum(c_acc[...], 1.0)
        g = jnp.maximum(_dot(g, w1_ref[...]) + b1_ref[...], 0.0)
        g = jnp.maximum(_dot(g, w2_ref[...]) + b2_ref[...], 0.0)
        lg = _dot(g, w3_ref[...]) + b3_ref[...]
        s = lg - jnp.max(lg, axis=1, keepdims=True)
        out_ref[...] = s - jnp.log(jnp.sum(jnp.exp(s), axis=1, keepdims=True))


def _row_spec(width):
    return pl.BlockSpec((BN, width), lambda i: (i, 0))


def _whole(shape):
    ndim = len(shape)
    return pl.BlockSpec(shape, lambda i: (0,) * ndim)


def _tc1(z, u0, u1, e0, e1, wx, v, wr, br):
    return pl.pallas_call(
        _tc1_body,
        grid=(GRID,),
        in_specs=[_row_spec(DZ), _row_spec(DZ), _row_spec(DZ),
                  _row_spec(DE), _row_spec(DE),
                  _whole((DF, H)), _whole((FW, H)), _whole((DF, H)),
                  _whole((1, H))],
        out_specs=[_row_spec(H), _row_spec(FW)],
        out_shape=[jax.ShapeDtypeStruct((NP, H), jnp.float32),
                   jax.ShapeDtypeStruct((NP, FW), jnp.float32)],
    )(z, u0, u1, e0, e1, wx, v, wr, br)


def _tc2(h, u0, u1, f, wx, v, wr, br):
    return pl.pallas_call(
        _tc2_body,
        grid=(GRID,),
        in_specs=[_row_spec(H), _row_spec(H), _row_spec(H), _row_spec(FW),
                  _whole((H, H)), _whole((FW, H)), _whole((H, H)),
                  _whole((1, H))],
        out_specs=_row_spec(H),
        out_shape=jax.ShapeDtypeStruct((NP, H), jnp.float32),
    )(h, u0, u1, f, wx, v, wr, br)


def _tc3(h, u0, u1, f, batchf, wx, v, wr, br, w1, b1, w2, b2, w3, b3):
    return pl.pallas_call(
        _tc3_body,
        grid=(GRID,),
        in_specs=[_row_spec(H), _row_spec(H), _row_spec(H), _row_spec(FW),
                  _row_spec(1),
                  _whole((H, H)), _whole((FW, H)), _whole((H, H)),
                  _whole((1, H)),
                  _whole((H, H)), _whole((1, H)), _whole((H, H)),
                  _whole((1, H)), _whole((H, 2)), _whole((1, 2))],
        out_specs=_whole((G, 2)),
        out_shape=jax.ShapeDtypeStruct((G, 2), jnp.float32),
        scratch_shapes=[pltpu.VMEM((G, H), jnp.float32),
                        pltpu.VMEM((G, H), jnp.float32)],
    )(h, u0, u1, f, batchf, wx, v, wr, br, w1, b1, w2, b2, w3, b3)


def _mk_v(wm, bm):
    return jnp.concatenate(
        [wm[DF:], bm[None, :], jnp.zeros((FW - DE - PD - 1, H), jnp.float32)],
        axis=0)


def kernel(x, edge_index, batch, edge_attr, pos,
           Wm1, bm1, Wr1, br1, Wm2, bm2, Wr2, br2, Wm3, bm3, Wr3, br3,
           W1, b1, W2, b2, W3, b3):
    f32 = jnp.float32
    padidx = (N + jnp.arange(EPP - E, dtype=jnp.int32) % (NP - N))
    src = jnp.concatenate([edge_index[0].astype(jnp.int32), padidx])
    dst = jnp.concatenate([edge_index[1].astype(jnp.int32), padidx])

    eatail = jnp.pad(edge_attr[(NW - 1) * EWP:], ((0, EAT - (E - (NW - 1) * EWP)), (0, 0)))

    z = jnp.concatenate(
        [x, pos, jnp.ones((N, 1), f32), jnp.zeros((N, DZ - DF - PD - 1), f32)],
        axis=1)
    z = jnp.pad(z, ((0, NP - N), (0, 0)))
    zz = jnp.zeros((NP, DZ), f32)
    zze = jnp.zeros((NP, DE), f32)
    zzh = jnp.zeros((NP, H), f32)
    batchf = jnp.pad(batch.astype(f32), (0, NP - N)).reshape(NP, 1)

    wx1, wx2, wx3 = Wm1[:DF], Wm2[:H], Wm3[:H]
    v1, v2, v3 = _mk_v(Wm1, bm1), _mk_v(Wm2, bm2), _mk_v(Wm3, bm3)
    br1r, br2r, br3r = br1[None, :], br2[None, :], br3[None, :]

    sc_agg1, sc_aggh, sc_ea = _sc_kernels()
    ue = sc_ea(edge_attr, eatail, dst, zze)
    uz = sc_agg1(z, src, dst, zz)
    h1, f = _tc1(z, uz[0], uz[1], ue[0], ue[1], wx1, v1, Wr1, br1r)
    u2 = sc_aggh(h1, src, dst, zzh)
    h2 = _tc2(h1, u2[0], u2[1], f, wx2, v2, Wr2, br2r)
    u3 = sc_aggh(h2, src, dst, zzh)
    return _tc3(h2, u3[0], u3[1], f, batchf, wx3, v3, Wr3, br3r,
                W1, b1[None, :], W2, b2[None, :], W3, b3[None, :])


# speedup vs baseline: 1.3372x; 1.1270x over previous
import jax
import jax.numpy as jnp
from jax import lax
from jax.experimental import pallas as pl
from jax.experimental.pallas import tpu as pltpu
from jax.experimental.pallas import tpu_sc as plsc

N = 10000
E = 320000
DF = 128
DE = 16
PD = 3
H = 128
G = 64

NP = 10240
DZ = 144
FW = 24
NC = 2
NS = 16
NW = NC * NS
EWP = 10240
EP = NW * EWP
CHA = 64
CH = 128
EPP = EP + 3 * CH
RPT = NP // NS
EAT = EWP + 3 * CH
BN = 1024
GRID = NP // BN


def _make_sc_body(ch):
    cpw = EWP // ch

    def body(z_hbm, src_hbm, dst_hbm, zz_hbm, uz_out,
             is0, is1, is2, is3, id0, id1, id2, id3,
             rows0, rows1, rows2, rows3, acc,
             sg0, sg1, sg2, sg3, ss0, ss1, ss2, ss3, si0, si1, si2, si3):
        cid = lax.axis_index("c")
        sid = lax.axis_index("s")
        wid = sid * NC + cid
        base = sid * RPT
        pltpu.sync_copy(zz_hbm.at[pl.ds(base, RPT)], acc.at[pl.ds(base, RPT)])
        ebase = wid * EWP
        idxs = (is0, is1, is2, is3)
        idxd = (id0, id1, id2, id3)
        rows = (rows0, rows1, rows2, rows3)
        sg = (sg0, sg1, sg2, sg3)
        ss = (ss0, ss1, ss2, ss3)
        si = (si0, si1, si2, si3)

        def eoff(j):
            return pl.multiple_of(ebase + j * ch, 8)

        def idx_start(j, b):
            pltpu.async_copy(src_hbm.at[pl.ds(eoff(j), ch)], idxs[b], si[b])
            pltpu.async_copy(dst_hbm.at[pl.ds(eoff(j), ch)], idxd[b], si[b])

        def idx_wait(j, b):
            pltpu.make_async_copy(src_hbm.at[pl.ds(eoff(j), ch)],
                                  idxs[b], si[b]).wait()
            pltpu.make_async_copy(dst_hbm.at[pl.ds(eoff(j), ch)],
                                  idxd[b], si[b]).wait()

        def gather_start(j, b):
            pltpu.async_copy(z_hbm.at[idxs[b]], rows[b], sg[b])

        def gather_wait(j, b):
            pltpu.make_async_copy(z_hbm.at[idxs[b]], rows[b], sg[b]).wait()

        def scatter_start(j, b):
            pltpu.async_copy(rows[b], acc.at[idxd[b]], ss[b], add=True)

        def scatter_wait(j, b):
            pltpu.make_async_copy(rows[b], acc.at[idxd[b]], ss[b]).wait()

        def step(j, b, skip_sw=False):
            if not skip_sw:
                scatter_wait(j - 1, (b - 1) % 4)
            gather_wait(j, b)
            idx_wait(j + 2, (b + 2) % 4)
            gather_start(j + 2, (b + 2) % 4)
            scatter_start(j, b)
            idx_start(j + 3, (b + 3) % 4)

        pltpu.sync_copy(src_hbm.at[pl.ds(eoff(0), ch)], idxs[0])
        pltpu.sync_copy(dst_hbm.at[pl.ds(eoff(0), ch)], idxd[0])
        idx_start(1, 1)
        idx_start(2, 2)
        idx_wait(1, 1)
        gather_start(0, 0)
        gather_start(1, 1)
        step(0, 0, skip_sw=True)
        step(1, 1)
        step(2, 2)
        step(3, 3)

        def group(i, carry):
            j = 4 * i
            step(j, 0)
            step(j + 1, 1)
            step(j + 2, 2)
            step(j + 3, 3)
            return carry

        lax.fori_loop(1, cpw // 4, group, 0)
        scatter_wait(cpw - 1, (cpw - 1) % 4)
        gather_wait(cpw, cpw % 4)
        gather_wait(cpw + 1, (cpw + 1) % 4)
        idx_wait(cpw + 2, (cpw + 2) % 4)
        plsc.subcore_barrier()
        pltpu.sync_copy(acc.at[pl.ds(base, RPT)],
                        uz_out.at[cid, pl.ds(base, RPT)])

    return body


def _sc_ea_body(ea_hbm, eat_hbm, dst_hbm, zze_hbm, ue_out,
                id0, id1, ea0, ea1, acce,
                sg0, sg1, si0, si1):
    cid = lax.axis_index("c")
    sid = lax.axis_index("s")
    wid = sid * NC + cid
    base = sid * RPT
    pltpu.sync_copy(zze_hbm.at[pl.ds(base, RPT)], acce.at[pl.ds(base, RPT)])
    ebase = wid * EWP
    idxd = (id0, id1)
    eab = (ea0, ea1)
    sg = (sg0, sg1)
    si = (si0, si1)
    last = wid == NW - 1

    def eoff(j):
        return pl.multiple_of(ebase + j * CH, 8)

    def toff(j):
        return pl.multiple_of(j * CH, 8)

    def idx_start(j, b):
        pltpu.async_copy(dst_hbm.at[pl.ds(eoff(j), CH)], idxd[b], si[b])

    def idx_wait(j, b):
        pltpu.make_async_copy(dst_hbm.at[pl.ds(eoff(j), CH)],
                              idxd[b], si[b]).wait()

    def ea_start(j, b):
        @pl.when(last)
        def _():
            pltpu.async_copy(eat_hbm.at[pl.ds(toff(j), CH)], eab[b], sg[b])

        @pl.when(jnp.logical_not(last))
        def _():
            pltpu.async_copy(ea_hbm.at[pl.ds(eoff(j), CH)], eab[b], sg[b])

    def ea_wait(j, b):
        @pl.when(last)
        def _():
            pltpu.make_async_copy(eat_hbm.at[pl.ds(toff(j), CH)],
                                  eab[b], sg[b]).wait()

        @pl.when(jnp.logical_not(last))
        def _():
            pltpu.make_async_copy(ea_hbm.at[pl.ds(eoff(j), CH)],
                                  eab[b], sg[b]).wait()

    def scatter(j, b):
        pltpu.sync_copy(eab[b], acce.at[idxd[b]], add=True)

    pltpu.sync_copy(dst_hbm.at[pl.ds(eoff(0), CH)], idxd[0])
    ea_start(0, 0)
    idx_start(1, 1)

    def step(j, b):
        ea_wait(j, b)
        idx_wait(j + 1, 1 - b)
        ea_start(j + 1, 1 - b)
        scatter(j, b)
        idx_start(j + 2, b)

    def pair(i, carry):
        step(2 * i, 0)
        step(2 * i + 1, 1)
        return carry

    lax.fori_loop(0, EWP // CH // 2, pair, 0)
    ea_wait(EWP // CH, 0)
    idx_wait(EWP // CH + 1, 1)
    plsc.subcore_barrier()
    pltpu.sync_copy(acce.at[pl.ds(base, RPT)],
                    ue_out.at[cid, pl.ds(base, RPT)])


import functools


@functools.cache
def _sc_kernels():
    mesh = plsc.VectorSubcoreMesh(core_axis_name="c", subcore_axis_name="s")
    params = pltpu.CompilerParams(use_tc_tiling_on_sc=False)
    sems = [pltpu.SemaphoreType.DMA] * 4

    def _agg_scratch(width):
        return ([pltpu.VMEM((CHA,), jnp.int32)] * 8
                + [pltpu.VMEM((CHA, width), jnp.float32)] * 4
                + [pltpu.VMEM_SHARED((NP, width), jnp.float32)]
                + [pltpu.SemaphoreType.DMA] * 12)

    sc_agg1 = pl.kernel(
        _make_sc_body(CHA),
        mesh=mesh,
        compiler_params=params,
        out_type=jax.ShapeDtypeStruct((NC, NP, DZ), jnp.float32),
        scratch_types=_agg_scratch(DZ),
    )
    sc_aggh = pl.kernel(
        _make_sc_body(CHA),
        mesh=mesh,
        out_type=jax.ShapeDtypeStruct((NC, NP, H), jnp.float32),
        scratch_types=_agg_scratch(H),
    )
    sc_ea = pl.kernel(
        _sc_ea_body,
        mesh=mesh,
        out_type=jax.ShapeDtypeStruct((NC, NP, DE), jnp.float32),
        scratch_types=[
            pltpu.VMEM((CH,), jnp.int32),
            pltpu.VMEM((CH,), jnp.int32),
            pltpu.VMEM((CH, DE), jnp.float32),
            pltpu.VMEM((CH, DE), jnp.float32),
            pltpu.VMEM_SHARED((NP, DE), jnp.float32),
        ] + sems,
    )
    return sc_agg1, sc_aggh, sc_ea


def _dot(a, b):
    return jnp.dot(a, b, preferred_element_type=jnp.float32)


def _tc1_body(z_ref, u0_ref, u1_ref, e0_ref, e1_ref,
              wx_ref, v_ref, wr_ref, br_ref,
              h1_ref, f_ref):
    u = u0_ref[...] + u1_ref[...]
    se = e0_ref[...] + e1_ref[...]
    xb = z_ref[:, :DF]
    posb = z_ref[:, DF:DF + PD]
    deg = u[:, DF + PD:DF + PD + 1]
    invd = 1.0 / jnp.maximum(deg, 1.0)
    m = jnp.where(deg > 0, 1.0, 0.0)
    sx = u[:, :DF]
    sp = u[:, DF:DF + PD]
    zero3 = jnp.zeros_like(posb)
    f = jnp.concatenate(
        [se * invd, (sp - deg * posb) * invd, m, invd, zero3], axis=1)
    agg = _dot(sx, wx_ref[...]) * invd + _dot(f, v_ref[...])
    h1_ref[...] = jnp.maximum(_dot(xb, wr_ref[...]) + br_ref[...] + agg, 0.0)
    f_ref[...] = f


def _tc2_body(h_ref, u0_ref, u1_ref, f_ref,
              wx_ref, v_ref, wr_ref, br_ref,
              ho_ref):
    u = u0_ref[...] + u1_ref[...]
    f = f_ref[...]
    agg = _dot(u, wx_ref[...]) * f[:, FW - 4:FW - 3] + _dot(f, v_ref[...])
    ho_ref[...] = jnp.maximum(
        _dot(h_ref[...], wr_ref[...]) + br_ref[...] + agg, 0.0)


def _tc3_body(h_ref, u0_ref, u1_ref, f_ref, batch_ref,
              wx_ref, v_ref, wr_ref, br_ref,
              w1_ref, b1_ref, w2_ref, b2_ref, w3_ref, b3_ref,
              out_ref, p_acc, c_acc):
    i = pl.program_id(0)
    u = u0_ref[...] + u1_ref[...]
    f = f_ref[...]
    agg = _dot(u, wx_ref[...]) * f[:, FW - 4:FW - 3] + _dot(f, v_ref[...])
    h3 = jnp.maximum(_dot(h_ref[...], wr_ref[...]) + br_ref[...] + agg, 0.0)

    @pl.when(i == 0)
    def _():
        p_acc[...] = jnp.zeros_like(p_acc)
        c_acc[...] = jnp.zeros_like(c_acc)

    rowid = i * BN + lax.broadcasted_iota(jnp.int32, (BN, 1), 0)
    valid = rowid < N
    bb = batch_ref[...]
    gid = lax.broadcasted_iota(jnp.int32, (1, G), 1).astype(jnp.float32)
    onehot = jnp.where((bb == gid) & valid, 1.0, 0.0)
    dimnum = (((0,), (0,)), ((), ()))
    p_acc[...] += lax.dot_general(onehot, h3, dimnum,
                                  preferred_element_type=jnp.float32)
    c_acc[...] += lax.dot_general(onehot, jnp.ones_like(h3), dimnum,
                                  preferred_element_type=jnp.float32)

    @pl.when(i == GRID - 1)
    def _():
        g = p_acc[...] / jnp.maximum(c_acc[...], 1.0)
        g = jnp.maximum(_dot(g, w1_ref[...]) + b1_ref[...], 0.0)
        g = jnp.maximum(_dot(g, w2_ref[...]) + b2_ref[...], 0.0)
        lg = _dot(g, w3_ref[...]) + b3_ref[...]
        s = lg - jnp.max(lg, axis=1, keepdims=True)
        out_ref[...] = s - jnp.log(jnp.sum(jnp.exp(s), axis=1, keepdims=True))


def _row_spec(width):
    return pl.BlockSpec((BN, width), lambda i: (i, 0))


def _whole(shape):
    ndim = len(shape)
    return pl.BlockSpec(shape, lambda i: (0,) * ndim)


def _tc1(z, u0, u1, e0, e1, wx, v, wr, br):
    return pl.pallas_call(
        _tc1_body,
        grid=(GRID,),
        in_specs=[_row_spec(DZ), _row_spec(DZ), _row_spec(DZ),
                  _row_spec(DE), _row_spec(DE),
                  _whole((DF, H)), _whole((FW, H)), _whole((DF, H)),
                  _whole((1, H))],
        out_specs=[_row_spec(H), _row_spec(FW)],
        out_shape=[jax.ShapeDtypeStruct((NP, H), jnp.float32),
                   jax.ShapeDtypeStruct((NP, FW), jnp.float32)],
    )(z, u0, u1, e0, e1, wx, v, wr, br)


def _tc2(h, u0, u1, f, wx, v, wr, br):
    return pl.pallas_call(
        _tc2_body,
        grid=(GRID,),
        in_specs=[_row_spec(H), _row_spec(H), _row_spec(H), _row_spec(FW),
                  _whole((H, H)), _whole((FW, H)), _whole((H, H)),
                  _whole((1, H))],
        out_specs=_row_spec(H),
        out_shape=jax.ShapeDtypeStruct((NP, H), jnp.float32),
    )(h, u0, u1, f, wx, v, wr, br)


def _tc3(h, u0, u1, f, batchf, wx, v, wr, br, w1, b1, w2, b2, w3, b3):
    return pl.pallas_call(
        _tc3_body,
        grid=(GRID,),
        in_specs=[_row_spec(H), _row_spec(H), _row_spec(H), _row_spec(FW),
                  _row_spec(1),
                  _whole((H, H)), _whole((FW, H)), _whole((H, H)),
                  _whole((1, H)),
                  _whole((H, H)), _whole((1, H)), _whole((H, H)),
                  _whole((1, H)), _whole((H, 2)), _whole((1, 2))],
        out_specs=_whole((G, 2)),
        out_shape=jax.ShapeDtypeStruct((G, 2), jnp.float32),
        scratch_shapes=[pltpu.VMEM((G, H), jnp.float32),
                        pltpu.VMEM((G, H), jnp.float32)],
    )(h, u0, u1, f, batchf, wx, v, wr, br, w1, b1, w2, b2, w3, b3)


def _mk_v(wm, bm):
    return jnp.concatenate(
        [wm[DF:], bm[None, :], jnp.zeros((FW - DE - PD - 1, H), jnp.float32)],
        axis=0)


def kernel(x, edge_index, batch, edge_attr, pos,
           Wm1, bm1, Wr1, br1, Wm2, bm2, Wr2, br2, Wm3, bm3, Wr3, br3,
           W1, b1, W2, b2, W3, b3):
    f32 = jnp.float32
    padidx = (N + jnp.arange(EPP - E, dtype=jnp.int32) % (NP - N))
    src = jnp.concatenate([edge_index[0].astype(jnp.int32), padidx])
    dst = jnp.concatenate([edge_index[1].astype(jnp.int32), padidx])

    eatail = jnp.pad(edge_attr[(NW - 1) * EWP:], ((0, EAT - (E - (NW - 1) * EWP)), (0, 0)))

    z = jnp.concatenate(
        [x, pos, jnp.ones((N, 1), f32), jnp.zeros((N, DZ - DF - PD - 1), f32)],
        axis=1)
    z = jnp.pad(z, ((0, NP - N), (0, 0)))
    zz = jnp.zeros((NP, DZ), f32)
    zze = jnp.zeros((NP, DE), f32)
    zzh = jnp.zeros((NP, H), f32)
    batchf = jnp.pad(batch.astype(f32), (0, NP - N)).reshape(NP, 1)

    wx1, wx2, wx3 = Wm1[:DF], Wm2[:H], Wm3[:H]
    v1, v2, v3 = _mk_v(Wm1, bm1), _mk_v(Wm2, bm2), _mk_v(Wm3, bm3)
    br1r, br2r, br3r = br1[None, :], br2[None, :], br3[None, :]

    sc_agg1, sc_aggh, sc_ea = _sc_kernels()
    ue = sc_ea(edge_attr, eatail, dst, zze)
    uz = sc_agg1(z, src, dst, zz)
    h1, f = _tc1(z, uz[0], uz[1], ue[0], ue[1], wx1, v1, Wr1, br1r)
    u2 = sc_aggh(h1, src, dst, zzh)
    h2 = _tc2(h1, u2[0], u2[1], f, wx2, v2, Wr2, br2r)
    u3 = sc_aggh(h2, src, dst, zzh)
    return _tc3(h2, u3[0], u3[1], f, batchf, wx3, v3, Wr3, br3r,
                W1, b1[None, :], W2, b2[None, :], W3, b3[None, :])
